# Initial kernel scaffold; baseline (speedup 1.0000x reference)
import jax
import jax.numpy as jnp
from jax.experimental import pallas as pl


def kernel(token_ids, table, W, b):
    raise NotImplementedError("write your pallas kernel here")


# speedup vs baseline: 8.0121x; 8.0121x over previous
import functools

import jax
import jax.numpy as jnp
from jax import lax
from jax.experimental import pallas as pl
from jax.experimental.pallas import tpu as pltpu
from jax.experimental.pallas import tpu_sc as plsc

DIM = 64
NB = 16384
SEQ = 100
NCORE = 2
NSUB = 16
NWORK = NCORE * NSUB
DPW = NB // NWORK
LANES = 16
NCH = DIM // LANES


def _sc_pool(token_ids, table):
    mesh = plsc.VectorSubcoreMesh(core_axis_name="c", subcore_axis_name="s")

    @functools.partial(
        pl.kernel,
        out_type=jax.ShapeDtypeStruct((NB, DIM), jnp.float32),
        mesh=mesh,
        scratch_types=[
            pltpu.VMEM((DPW, SEQ), jnp.int32),
            pltpu.VMEM((SEQ, DIM), jnp.float32),
            pltpu.VMEM((DPW, DIM), jnp.float32),
            pltpu.SemaphoreType.DMA,
        ],
        compiler_params=pltpu.CompilerParams(use_tc_tiling_on_sc=False),
    )
    def pool(tok_hbm, table_hbm, out_hbm, idx_v, rows_v, acc_v, sem):
        wid = lax.axis_index("s") * NCORE + lax.axis_index("c")
        base = wid * DPW
        pltpu.sync_copy(tok_hbm.at[pl.ds(base, DPW), :], idx_v)

        def doc_body(d, carry):
            pltpu.async_copy(table_hbm.at[idx_v.at[d]], rows_v, sem).wait()

            def red(r, accs):
                return tuple(
                    accs[c] + rows_v[r, pl.ds(c * LANES, LANES)]
                    for c in range(NCH)
                )

            accs = lax.fori_loop(
                0, SEQ, red,
                tuple(jnp.zeros((LANES,), jnp.float32) for _ in range(NCH)),
            )
            for c in range(NCH):
                acc_v[d, pl.ds(c * LANES, LANES)] = accs[c]
            return carry

        lax.fori_loop(0, DPW, doc_body, 0)
        pltpu.sync_copy(acc_v, out_hbm.at[pl.ds(base, DPW), :])

    return pool(token_ids, table)


def _tc_proj(sums, W, b):
    blk = 2048

    def proj(s_ref, w_ref, b_ref, o_ref):
        o_ref[...] = (
            lax.dot_general(
                s_ref[...], w_ref[...], (((1,), (1,)), ((), ())),
                preferred_element_type=jnp.float32,
            ) * (1.0 / SEQ)
            + b_ref[...]
        )

    return pl.pallas_call(
        proj,
        grid=(NB // blk,),
        in_specs=[
            pl.BlockSpec((blk, DIM), lambda i: (i, 0)),
            pl.BlockSpec((DIM, DIM), lambda i: (0, 0)),
            pl.BlockSpec((1, DIM), lambda i: (0, 0)),
        ],
        out_specs=pl.BlockSpec((blk, DIM), lambda i: (i, 0)),
        out_shape=jax.ShapeDtypeStruct((NB, DIM), jnp.float32),
    )(sums, W, b.reshape(1, DIM))


@jax.jit
def kernel(token_ids, table, W, b):
    sums = _sc_pool(token_ids, table)
    return _tc_proj(sums, W, b)


# speedup vs baseline: 12.2774x; 1.5324x over previous
import functools

import jax
import jax.numpy as jnp
from jax import lax
from jax.experimental import pallas as pl
from jax.experimental.pallas import tpu as pltpu
from jax.experimental.pallas import tpu_sc as plsc

DIM = 64
NB = 16384
SEQ = 100
NCORE = 2
NSUB = 16
NWORK = NCORE * NSUB
DPW = NB // NWORK
LANES = 16
NCH = DIM // LANES
GK = 4
HALF = DPW // 2
NGRP = HALF // GK
RUN = 4


def _sc_pool(token_ids, table):
    mesh = plsc.VectorSubcoreMesh(core_axis_name="c", subcore_axis_name="s")

    @functools.partial(
        pl.kernel,
        out_type=jax.ShapeDtypeStruct((NB, DIM), jnp.float32),
        mesh=mesh,
        scratch_types=[
            pltpu.VMEM((HALF, SEQ), jnp.int32),
            pltpu.VMEM((2 * GK, SEQ, DIM), jnp.float32),
            pltpu.VMEM((DPW, DIM), jnp.float32),
            pltpu.SemaphoreType.DMA,
            pltpu.SemaphoreType.DMA,
        ],
        compiler_params=pltpu.CompilerParams(use_tc_tiling_on_sc=False),
    )
    def pool(tok_hbm, table_hbm, out_hbm, idx_v, rows_v, acc_v, sem_a, sem_b):
        wid = lax.axis_index("s") * NCORE + lax.axis_index("c")
        base = wid * DPW

        def fire(g, slot0, sem):
            for i in range(GK):
                pltpu.async_copy(
                    table_hbm.at[idx_v.at[g * GK + i]], rows_v.at[slot0 + i], sem
                )

        def drain(g, slot0, sem):
            for i in range(GK):
                pltpu.make_async_copy(
                    table_hbm.at[idx_v.at[g * GK + i]], rows_v.at[slot0 + i], sem
                ).wait()

        def reduce_group(g, slot0, acc_base):
            for i in range(GK):
                slot = slot0 + i

                def red(r, accs, slot=slot):
                    out = list(accs)
                    for rr in range(RUN):
                        row = r * RUN + rr
                        for c in range(NCH):
                            out[c] = out[c] + rows_v[
                                slot, row, pl.ds(c * LANES, LANES)
                            ]
                    return tuple(out)

                accs = lax.fori_loop(
                    0, SEQ // RUN, red,
                    tuple(jnp.zeros((LANES,), jnp.float32) for _ in range(NCH)),
                )
                for c in range(NCH):
                    acc_v[acc_base + g * GK + i, pl.ds(c * LANES, LANES)] = accs[c]

        for h in range(2):
            hbase = base + h * HALF
            pltpu.sync_copy(tok_hbm.at[pl.ds(hbase, HALF), :], idx_v)
            fire(0, 0, sem_a)

            def jj_body(jj, carry, h=h):
                g = 2 * jj
                fire(g + 1, GK, sem_b)
                drain(g, 0, sem_a)
                reduce_group(g, 0, h * HALF)

                @pl.when(g + 2 < NGRP)
                def _():
                    fire(g + 2, 0, sem_a)

                drain(g + 1, GK, sem_b)
                reduce_group(g + 1, GK, h * HALF)
                return carry

            lax.fori_loop(0, NGRP // 2, jj_body, 0)

        pltpu.sync_copy(acc_v, out_hbm.at[pl.ds(base, DPW), :])

    return pool(token_ids, table)


def _tc_proj(sums, W, b):
    blk = 2048

    def proj(s_ref, w_ref, b_ref, o_ref):
        o_ref[...] = (
            lax.dot_general(
                s_ref[...], w_ref[...], (((1,), (1,)), ((), ())),
                preferred_element_type=jnp.float32,
            ) * (1.0 / SEQ)
            + b_ref[...]
        )

    return pl.pallas_call(
        proj,
        grid=(NB // blk,),
        in_specs=[
            pl.BlockSpec((blk, DIM), lambda i: (i, 0)),
            pl.BlockSpec((DIM, DIM), lambda i: (0, 0)),
            pl.BlockSpec((1, DIM), lambda i: (0, 0)),
        ],
        out_specs=pl.BlockSpec((blk, DIM), lambda i: (i, 0)),
        out_shape=jax.ShapeDtypeStruct((NB, DIM), jnp.float32),
    )(sums, W, b.reshape(1, DIM))


@jax.jit
def kernel(token_ids, table, W, b):
    sums = _sc_pool(token_ids, table)
    return _tc_proj(sums, W, b)
